# baseline (speedup 1.0000x reference)
import functools
import jax
import jax.numpy as jnp
from jax import lax
from jax.experimental import pallas as pl
from jax.experimental.pallas import tpu as pltpu
from jax.experimental.pallas import tpu_sc as plsc

B = 64
R = 65536
NW = 32
RW = R // NW
C = 512
NCH = RW // C


def _splat(v):
    return jnp.full((16,), v, jnp.int32)


def _sc_body(x_hbm, mf_hbm, out_hbm, xv, mfv, tlo, thi, pv, qv, outv0, outv1,
             sem0, sem1):
    wid = lax.axis_index("s") * 2 + lax.axis_index("c")
    base = wid * RW

    pltpu.sync_copy(x_hbm, xv)
    pltpu.sync_copy(mf_hbm.at[pl.ds(base, RW)], mfv)

    iota = lax.iota(jnp.int32, 16)
    hi2 = iota >> 2
    lo2 = iota & 3

    @plsc.parallel_loop(0, B, step=1, unroll=4)
    def build(b):
        xb = b * 32
        p01 = (plsc.load_gather(xv, [xb + hi2])
               * plsc.load_gather(xv, [xb + 4 + lo2]))
        p23 = (plsc.load_gather(xv, [xb + 8 + hi2])
               * plsc.load_gather(xv, [xb + 12 + lo2]))
        p45 = (plsc.load_gather(xv, [xb + 16 + hi2])
               * plsc.load_gather(xv, [xb + 20 + lo2]))
        p67 = (plsc.load_gather(xv, [xb + 24 + hi2])
               * plsc.load_gather(xv, [xb + 28 + lo2]))
        pv[b, pl.ds(16, 16)] = p01
        qv[b, pl.ds(16, 16)] = p45
        tb = b * 256
        bsplat = jnp.full((16,), 0, jnp.int32) + b
        for j in range(16):
            s_lo = plsc.load_gather(pv, [bsplat, _splat(16 + j)])
            tlo[pl.ds(tb + j * 16, 16)] = s_lo * p23
            s_hi = plsc.load_gather(qv, [bsplat, _splat(16 + j)])
            thi[pl.ds(tb + j * 16, 16)] = s_hi * p67

    def run_chunk(outbuf, cbase):
        @plsc.parallel_loop(0, C // 16, unroll=1)
        def rv_body(rv):
            code = mfv[pl.ds(cbase + rv * 16, 16)]
            ilo = (code >> 8) & 255
            ihi = code & 255
            cb = rv >> 3
            col = (rv & 7) * 16

            @plsc.parallel_loop(0, B, step=1, unroll=32)
            def b_body(b):
                vlo = plsc.load_gather(tlo, [ilo + b * 256])
                vhi = plsc.load_gather(thi, [ihi + b * 256])
                outbuf[b >> 3, cb, b & 7, pl.ds(col, 16)] = vlo * vhi

    pending = [None, None]
    for c in range(NCH):
        buf, sem = (outv0, sem0) if c % 2 == 0 else (outv1, sem1)
        if pending[c % 2] is not None:
            pending[c % 2].wait()
        run_chunk(buf, c * C)
        cb0 = (base + c * C) // 128
        pending[c % 2] = pltpu.async_copy(
            buf, out_hbm.at[:, pl.ds(cb0, C // 128)], sem
        )
    pending[0].wait()
    pending[1].wait()


@jax.jit
def _run(xf, mff):
    f = functools.partial(
        pl.kernel,
        out_type=jax.ShapeDtypeStruct((B // 8, R // 128, 8, 128), jnp.float32),
        mesh=plsc.VectorSubcoreMesh(core_axis_name="c", subcore_axis_name="s"),
        compiler_params=pltpu.CompilerParams(
            needs_layout_passes=False, use_tc_tiling_on_sc=False
        ),
        scratch_types=[
            pltpu.VMEM((B * 32,), jnp.float32),
            pltpu.VMEM((RW,), jnp.int32),
            pltpu.VMEM((B * 256,), jnp.float32),
            pltpu.VMEM((B * 256,), jnp.float32),
            pltpu.VMEM((B, 32), jnp.float32),
            pltpu.VMEM((B, 32), jnp.float32),
            pltpu.VMEM((B // 8, C // 128, 8, 128), jnp.float32),
            pltpu.VMEM((B // 8, C // 128, 8, 128), jnp.float32),
            pltpu.SemaphoreType.DMA,
            pltpu.SemaphoreType.DMA,
        ],
    )(_sc_body)
    out4 = f(xf, mff)
    return out4.transpose(0, 2, 1, 3).reshape(B, R)


def kernel(x, mf_idx):
    shifts = jnp.arange(14, -2, -2, dtype=jnp.int32)
    code = jnp.sum(mf_idx.astype(jnp.int32) << shifts[None, :], axis=1,
                   dtype=jnp.int32)
    return _run(x.reshape(-1), code)


# speedup vs baseline: 1.0350x; 1.0350x over previous
import functools
import jax
import jax.numpy as jnp
from jax import lax
from jax.experimental import pallas as pl
from jax.experimental.pallas import tpu as pltpu
from jax.experimental.pallas import tpu_sc as plsc

B = 64
R = 65536
NW = 32
RW = R // NW
C = 512
NCH = RW // C


def _splat(v):
    return jnp.full((16,), v, jnp.int32)


def _sc_body(x_hbm, mf_hbm, out_hbm, xv, mfv, tlo, thi, pv, qv, outv0, outv1,
             sem0, sem1):
    wid = lax.axis_index("s") * 2 + lax.axis_index("c")
    base = wid * RW

    pltpu.sync_copy(x_hbm, xv)
    pltpu.sync_copy(mf_hbm.at[pl.ds(base, RW)], mfv)

    iota = lax.iota(jnp.int32, 16)
    hi2 = iota >> 2
    lo2 = iota & 3

    @plsc.parallel_loop(0, B, step=1, unroll=4)
    def build(b):
        xb = b * 32
        p01 = (plsc.load_gather(xv, [xb + hi2])
               * plsc.load_gather(xv, [xb + 4 + lo2]))
        p23 = (plsc.load_gather(xv, [xb + 8 + hi2])
               * plsc.load_gather(xv, [xb + 12 + lo2]))
        p45 = (plsc.load_gather(xv, [xb + 16 + hi2])
               * plsc.load_gather(xv, [xb + 20 + lo2]))
        p67 = (plsc.load_gather(xv, [xb + 24 + hi2])
               * plsc.load_gather(xv, [xb + 28 + lo2]))
        pv[b, pl.ds(16, 16)] = p01
        qv[b, pl.ds(16, 16)] = p45
        tb = b * 256
        bsplat = jnp.full((16,), 0, jnp.int32) + b
        for j in range(16):
            s_lo = plsc.load_gather(pv, [bsplat, _splat(16 + j)])
            tlo[pl.ds(tb + j * 16, 16)] = s_lo * p23
            s_hi = plsc.load_gather(qv, [bsplat, _splat(16 + j)])
            thi[pl.ds(tb + j * 16, 16)] = s_hi * p67

    def run_chunk(outbuf, cbase):
        @plsc.parallel_loop(0, C // 16, unroll=2)
        def rv_body(rv):
            code = mfv[pl.ds(cbase + rv * 16, 16)]
            ilo = (code >> 8) & 255
            ihi = code & 255
            cb = rv >> 3
            col = (rv & 7) * 16

            @plsc.parallel_loop(0, B, step=1, unroll=16)
            def b_body(b):
                vlo = plsc.load_gather(tlo, [ilo + b * 256])
                vhi = plsc.load_gather(thi, [ihi + b * 256])
                outbuf[b >> 3, cb, b & 7, pl.ds(col, 16)] = vlo * vhi

    pending = [None, None]
    for c in range(NCH):
        buf, sem = (outv0, sem0) if c % 2 == 0 else (outv1, sem1)
        if pending[c % 2] is not None:
            pending[c % 2].wait()
        run_chunk(buf, c * C)
        cb0 = (base + c * C) // 128
        pending[c % 2] = pltpu.async_copy(
            buf, out_hbm.at[:, pl.ds(cb0, C // 128)], sem
        )
    pending[0].wait()
    pending[1].wait()


@jax.jit
def _run(xf, mff):
    f = functools.partial(
        pl.kernel,
        out_type=jax.ShapeDtypeStruct((B // 8, R // 128, 8, 128), jnp.float32),
        mesh=plsc.VectorSubcoreMesh(core_axis_name="c", subcore_axis_name="s"),
        compiler_params=pltpu.CompilerParams(
            needs_layout_passes=False, use_tc_tiling_on_sc=False
        ),
        scratch_types=[
            pltpu.VMEM((B * 32,), jnp.float32),
            pltpu.VMEM((RW,), jnp.int32),
            pltpu.VMEM((B * 256,), jnp.float32),
            pltpu.VMEM((B * 256,), jnp.float32),
            pltpu.VMEM((B, 32), jnp.float32),
            pltpu.VMEM((B, 32), jnp.float32),
            pltpu.VMEM((B // 8, C // 128, 8, 128), jnp.float32),
            pltpu.VMEM((B // 8, C // 128, 8, 128), jnp.float32),
            pltpu.SemaphoreType.DMA,
            pltpu.SemaphoreType.DMA,
        ],
    )(_sc_body)
    out4 = f(xf, mff)
    return out4.transpose(0, 2, 1, 3).reshape(B, R)


def kernel(x, mf_idx):
    shifts = jnp.arange(14, -2, -2, dtype=jnp.int32)
    code = jnp.sum(mf_idx.astype(jnp.int32) << shifts[None, :], axis=1,
                   dtype=jnp.int32)
    return _run(x.reshape(-1), code)
